# baseline (speedup 1.0000x reference)
import functools

import jax
import jax.numpy as jnp
from jax import lax
from jax.experimental import pallas as pl
from jax.experimental.pallas import tpu as pltpu
from jax.experimental.pallas import tpu_sc as plsc

NN = 10000
NPAD = 10240
NE = 320000
D = 128
NG = 64
NC, NS = 2, 16
NT = NC * NS
EPT = NE // NT
CHUNK = 128
CPT = 80
NGROUP = CPT * CHUNK // 16
CPT_PAD = 96
RPS = NPAD // NS

_mesh = plsc.VectorSubcoreMesh(
    core_axis_name="c", subcore_axis_name="s", num_cores=NC, num_subcores=NS)


@functools.partial(
    pl.kernel,
    out_type=(
        jax.ShapeDtypeStruct((NT, CPT_PAD, CHUNK), jnp.int32),
        jax.ShapeDtypeStruct((NT, CPT, CHUNK), jnp.int32),
        jax.ShapeDtypeStruct((NT, NPAD), jnp.float32),
    ),
    mesh=_mesh,
    scratch_types=[
        pltpu.VMEM((EPT,), jnp.int32),
        pltpu.VMEM((EPT,), jnp.int32),
        pltpu.VMEM((CPT, CHUNK), jnp.int32),
        pltpu.VMEM((CPT, CHUNK), jnp.int32),
        pltpu.VMEM((NPAD,), jnp.float32),
        pltpu.VMEM((NPAD,), jnp.float32),
    ],
    compiler_params=pltpu.CompilerParams(needs_layout_passes=False),
)
def _prep(esrc_hbm, edst_hbm, src2_hbm, dst2_hbm, degp_hbm,
          src_v, dst_v, src2_v, dst2_v, deg_v, cnt_v):
    c = lax.axis_index("c")
    s = lax.axis_index("s")
    wid = s * NC + c
    base = wid * EPT
    pltpu.sync_copy(esrc_hbm.at[pl.ds(base, EPT)], src_v)
    pltpu.sync_copy(edst_hbm.at[pl.ds(base, EPT)], dst_v)

    def zero_body(i, carry):
        deg_v[pl.ds(i * 16, 16)] = jnp.zeros((16,), jnp.float32)
        cnt_v[pl.ds(i * 16, 16)] = jnp.zeros((16,), jnp.float32)
        return carry

    lax.fori_loop(0, NPAD // 16, zero_body, 0)

    lanes = lax.iota(jnp.int32, 16)

    def edges(g):
        eidx = g * 16 + lanes
        valid = eidx < EPT
        off = jnp.where(g * 16 < EPT, g * 16, 0)
        sv = src_v[pl.ds(off, 16)]
        dv = dst_v[pl.ds(off, 16)]
        keep = jnp.logical_and(valid, sv != dv)
        w = jnp.where(keep, jnp.float32(1.0), jnp.float32(0.0))
        d2 = jnp.where(valid, dv, jnp.int32(NN))
        s2 = jnp.where(keep, sv, jnp.int32(NN))
        return s2, d2, w

    def hist_body(g, carry):
        s2, d2, w = edges(g)
        plsc.addupdate_scatter(deg_v, [d2], w)
        plsc.addupdate_scatter(cnt_v, [s2], jnp.ones((16,), jnp.float32))
        return carry

    lax.fori_loop(0, NGROUP, hist_body, 0)

    def scan_body(i, runv):
        v = cnt_v[pl.ds(i * 16, 16)]
        cum = plsc.cumsum(v)
        cnt_v[pl.ds(i * 16, 16)] = cum - v + runv
        return runv + jnp.sum(v)

    lax.fori_loop(0, NPAD // 16, scan_body, jnp.zeros((16,), jnp.float32))

    def place_body(g, carry):
        s2, d2, _ = edges(g)
        basep = plsc.load_gather(cnt_v, [s2])
        rank, last = plsc.scan_count(s2)
        p = basep.astype(jnp.int32) + rank.astype(jnp.int32) - 1
        p = jnp.clip(p, 0, CPT * CHUNK - 1)
        plsc.store_scatter(src2_v, [p >> 7, p & 127], s2)
        plsc.store_scatter(dst2_v, [p >> 7, p & 127], d2)
        plsc.addupdate_scatter(
            cnt_v, [s2], rank.astype(jnp.float32), mask=last)
        return carry

    lax.fori_loop(0, NGROUP, place_body, 0)

    pltpu.sync_copy(src2_v, src2_hbm.at[wid, pl.ds(0, CPT)])
    pltpu.sync_copy(dst2_v, dst2_hbm.at[wid])
    pltpu.sync_copy(deg_v, degp_hbm.at[wid])


GRP = 8
NGRP = CPT // GRP


@functools.partial(
    pl.kernel,
    out_type=jax.ShapeDtypeStruct((NC, NPAD, D), jnp.float32),
    mesh=_mesh,
    scratch_types=[
        pltpu.VMEM((CPT, CHUNK), jnp.int32),
        pltpu.VMEM((GRP, CHUNK), jnp.int32),
        pltpu.VMEM((GRP, CHUNK), jnp.int32),
        pltpu.VMEM((CHUNK, D), jnp.float32),
        pltpu.VMEM((CHUNK, D), jnp.float32),
        pltpu.VMEM_SHARED((NPAD, D), jnp.float32),
        pltpu.SemaphoreType.DMA,
        pltpu.SemaphoreType.DMA,
        pltpu.SemaphoreType.DMA,
        pltpu.SemaphoreType.DMA,
        pltpu.SemaphoreType.DMA,
        pltpu.SemaphoreType.DMA,
    ],
)
def _spmm(m_hbm, src2_hbm, dst2_hbm, acc_hbm,
          dst2_v, idx_a, idx_b, rows_a, rows_b, acc_sh,
          sem_ga, sem_gb, sem_sa, sem_sb, sem_ia, sem_ib):
    c = lax.axis_index("c")
    s = lax.axis_index("s")
    wid = s * NC + c
    pltpu.sync_copy(dst2_hbm.at[wid], dst2_v)

    def zrow(i, carry):
        r = i // 8
        k = i - r * 8
        rows_a[r, pl.ds(k * 16, 16)] = jnp.zeros((16,), jnp.float32)
        return carry

    lax.fori_loop(0, CHUNK * 8, zrow, 0)
    row0 = s * RPS
    for t in range(RPS // CHUNK):
        pltpu.sync_copy(rows_a, acc_sh.at[pl.ds(row0 + t * CHUNK, CHUNK)])
    plsc.subcore_barrier()

    pltpu.async_copy(src2_hbm.at[wid, pl.ds(0, GRP)], idx_a, sem_ia)
    pltpu.async_copy(src2_hbm.at[wid, pl.ds(GRP, GRP)], idx_b, sem_ib)

    def run_group(g, idx_v, isem, next_off):
        pltpu.make_async_copy(
            src2_hbm.at[wid, pl.ds(0, GRP)], idx_v, isem).wait()
        dg = [None, None]
        ds_ = [None, None]
        bufs = (rows_a, rows_b)
        gsems = (sem_ga, sem_gb)
        ssems = (sem_sa, sem_sb)
        dg[0] = pltpu.async_copy(m_hbm.at[idx_v.at[0]], rows_a, sem_ga)
        dg[1] = pltpu.async_copy(m_hbm.at[idx_v.at[1]], rows_b, sem_gb)
        for i in range(GRP):
            p = i % 2
            dg[p].wait()
            ds_[p] = pltpu.async_copy(
                bufs[p], acc_sh.at[dst2_v.at[g * GRP + i]], ssems[p],
                add=True)
            if i + 2 < GRP:
                ds_[p].wait()
                dg[p] = pltpu.async_copy(
                    m_hbm.at[idx_v.at[i + 2]], bufs[p], gsems[p])
        pltpu.async_copy(src2_hbm.at[wid, pl.ds(next_off, GRP)], idx_v, isem)
        ds_[0].wait()
        ds_[1].wait()

    def pair_body(t, carry):
        run_group(2 * t, idx_a, sem_ia, (2 * t + 2) * GRP)
        run_group(2 * t + 1, idx_b, sem_ib, (2 * t + 3) * GRP)
        return carry

    lax.fori_loop(0, NGRP // 2, pair_body, 0)
    pltpu.make_async_copy(
        src2_hbm.at[wid, pl.ds(0, GRP)], idx_a, sem_ia).wait()
    pltpu.make_async_copy(
        src2_hbm.at[wid, pl.ds(0, GRP)], idx_b, sem_ib).wait()
    plsc.subcore_barrier()

    for t in range(RPS // CHUNK):
        pltpu.sync_copy(acc_sh.at[pl.ds(row0 + t * CHUNK, CHUNK)], rows_a)
        pltpu.sync_copy(rows_a, acc_hbm.at[c, pl.ds(row0 + t * CHUNK, CHUNK)])


def _dinv(degp):
    ones = jnp.ones((NT, 1), jnp.float32)
    deg = lax.dot_general(degp, ones, (((0,), (0,)), ((), ()))) + 1.0
    return lax.rsqrt(deg)


def _tc_first_body(degp_ref, x_ref, w_ref, out_ref):
    dinv = _dinv(degp_ref[...])
    g = lax.dot_general(x_ref[...], w_ref[...], (((1,), (1,)), ((), ())))
    out_ref[...] = g * dinv


_tc_first = pl.pallas_call(
    _tc_first_body, out_shape=jax.ShapeDtypeStruct((NPAD, D), jnp.float32))


def _tc_mid_body(degp_ref, acc_ref, m_ref, b_ref, w_ref, out_ref):
    dinv = _dinv(degp_ref[...])
    acc = acc_ref[0] + acc_ref[1]
    pre = dinv * (acc + m_ref[...]) + b_ref[...]
    rows = lax.broadcasted_iota(jnp.int32, (NPAD, 1), 0)
    h = jnp.where(rows < NN, jnp.maximum(pre, 0.0), 0.0)
    g = lax.dot_general(h, w_ref[...], (((1,), (1,)), ((), ())))
    out_ref[...] = g * dinv


_tc_mid = pl.pallas_call(
    _tc_mid_body, out_shape=jax.ShapeDtypeStruct((NPAD, D), jnp.float32))


def _tc_final_body(degp_ref, acc_ref, m_ref, b_ref, batch_ref, xin_ref,
                   wn_ref, bn_ref, wx_ref, bx_ref, out_ref):
    dinv = _dinv(degp_ref[...])
    acc = acc_ref[0] + acc_ref[1]
    pre = dinv * (acc + m_ref[...]) + b_ref[...]
    rows = lax.broadcasted_iota(jnp.int32, (NPAD, 1), 0)
    h = jnp.where(rows < NN, jnp.maximum(pre, 0.0), 0.0)

    cols = lax.broadcasted_iota(jnp.int32, (NPAD, NG), 1)
    p = jnp.where(batch_ref[...] == cols, jnp.float32(1.0), jnp.float32(0.0))
    sums = lax.dot_general(p, h, (((0,), (0,)), ((), ())))
    ones = jnp.ones((NPAD, 1), jnp.float32)
    cnt = lax.dot_general(p, ones, (((0,), (0,)), ((), ())))
    pooled = sums / jnp.maximum(cnt, 1.0)

    z = lax.dot_general(pooled, wn_ref[...], (((1,), (1,)), ((), ())))
    z = jnp.maximum(z + bn_ref[...], 0.0)
    a0 = jnp.sum(z * wx_ref[0:1, :], axis=1, keepdims=True) + bx_ref[0:1, 0:1]
    nn1 = jnp.sum(z * wx_ref[1:2, :], axis=1, keepdims=True) + bx_ref[0:1, 1:2]
    out_ref[...] = xin_ref[...] * (1.0 + nn1) - a0


_tc_final = pl.pallas_call(
    _tc_final_body, out_shape=jax.ShapeDtypeStruct((NG, 1), jnp.float32))


def kernel(x_in, x, edge_index, batch, W1, b1, W2, b2, Wn, bn, Wx, bx):
    x_pad = jnp.pad(x, ((0, NPAD - NN), (0, 0)))
    batch_pad = jnp.pad(batch, (0, NPAD - NN),
                        constant_values=NG).reshape(NPAD, 1)
    b1r = b1.reshape(1, D)
    b2r = b2.reshape(1, D)
    bnr = bn.reshape(1, D)
    bxr = bx.reshape(1, 2)

    src2, dst2, degp = _prep(edge_index[0], edge_index[1])
    m1 = _tc_first(degp, x_pad, W1)
    acc1 = _spmm(m1, src2, dst2)
    m2 = _tc_mid(degp, acc1, m1, b1r, W2)
    acc2 = _spmm(m2, src2, dst2)
    return _tc_final(degp, acc2, m2, b2r, batch_pad, x_in, Wn, bnr, Wx, bxr)


# speedup vs baseline: 1.0846x; 1.0846x over previous
import functools

import jax
import jax.numpy as jnp
from jax import lax
from jax.experimental import pallas as pl
from jax.experimental.pallas import tpu as pltpu
from jax.experimental.pallas import tpu_sc as plsc

NN = 10000
NPAD = 10240
NE = 320000
D = 128
NG = 64
NC, NS = 2, 16
NT = NC * NS
EPT = NE // NT
CHUNK = 128
CPT = 80
NGROUP = CPT * CHUNK // 16
CPT_PAD = 96
RPS = NPAD // NS

_mesh = plsc.VectorSubcoreMesh(
    core_axis_name="c", subcore_axis_name="s", num_cores=NC, num_subcores=NS)


@functools.partial(
    pl.kernel,
    out_type=(
        jax.ShapeDtypeStruct((NT, CPT_PAD, CHUNK), jnp.int32),
        jax.ShapeDtypeStruct((NT, CPT, CHUNK), jnp.int32),
        jax.ShapeDtypeStruct((NT, NPAD), jnp.float32),
    ),
    mesh=_mesh,
    scratch_types=[
        pltpu.VMEM((EPT,), jnp.int32),
        pltpu.VMEM((EPT,), jnp.int32),
        pltpu.VMEM((CPT, CHUNK), jnp.int32),
        pltpu.VMEM((CPT, CHUNK), jnp.int32),
        pltpu.VMEM((NPAD,), jnp.float32),
    ],
    compiler_params=pltpu.CompilerParams(needs_layout_passes=False),
)
def _prep(esrc_hbm, edst_hbm, src2_hbm, dst2_hbm, degp_hbm,
          src_v, dst_v, src2_v, dst2_v, deg_v):
    c = lax.axis_index("c")
    s = lax.axis_index("s")
    wid = s * NC + c
    base = wid * EPT
    pltpu.sync_copy(esrc_hbm.at[pl.ds(base, EPT)], src_v)
    pltpu.sync_copy(edst_hbm.at[pl.ds(base, EPT)], dst_v)

    def zero_body(i, carry):
        deg_v[pl.ds(i * 16, 16)] = jnp.zeros((16,), jnp.float32)
        return carry

    lax.fori_loop(0, NPAD // 16, zero_body, 0)

    lanes = lax.iota(jnp.int32, 16)

    def grp_body(g, carry):
        jc = g // 8
        k = g - jc * 8
        eidx = g * 16 + lanes
        valid = eidx < EPT
        off = jnp.where(g * 16 < EPT, g * 16, 0)
        sv = src_v[pl.ds(off, 16)]
        dv = dst_v[pl.ds(off, 16)]
        keep = jnp.logical_and(valid, sv != dv)
        w = jnp.where(keep, jnp.float32(1.0), jnp.float32(0.0))
        d2 = jnp.where(valid, dv, jnp.int32(NN))
        s2 = jnp.where(keep, sv, jnp.int32(NN))
        plsc.addupdate_scatter(deg_v, [d2], w)
        src2_v[jc, pl.ds(k * 16, 16)] = s2
        dst2_v[jc, pl.ds(k * 16, 16)] = d2
        return carry

    lax.fori_loop(0, NGROUP, grp_body, 0)

    pltpu.sync_copy(src2_v, src2_hbm.at[wid, pl.ds(0, CPT)])
    pltpu.sync_copy(dst2_v, dst2_hbm.at[wid])
    pltpu.sync_copy(deg_v, degp_hbm.at[wid])


GRP = 8
NGRP = CPT // GRP
CH2 = 64
NSUB = GRP * CHUNK // CH2


@functools.partial(
    pl.kernel,
    out_type=jax.ShapeDtypeStruct((NC, NPAD, D), jnp.float32),
    mesh=_mesh,
    scratch_types=[
        pltpu.VMEM((CPT, CHUNK), jnp.int32),
        pltpu.VMEM((GRP, CHUNK), jnp.int32),
        pltpu.VMEM((GRP, CHUNK), jnp.int32),
        pltpu.VMEM((CH2, D), jnp.float32),
        pltpu.VMEM((CH2, D), jnp.float32),
        pltpu.VMEM((CH2, D), jnp.float32),
        pltpu.VMEM((CH2, D), jnp.float32),
        pltpu.VMEM_SHARED((NPAD, D), jnp.float32),
        pltpu.SemaphoreType.DMA,
        pltpu.SemaphoreType.DMA,
        pltpu.SemaphoreType.DMA,
        pltpu.SemaphoreType.DMA,
        pltpu.SemaphoreType.DMA,
        pltpu.SemaphoreType.DMA,
        pltpu.SemaphoreType.DMA,
        pltpu.SemaphoreType.DMA,
        pltpu.SemaphoreType.DMA,
        pltpu.SemaphoreType.DMA,
    ],
)
def _spmm(m_hbm, src2_hbm, dst2_hbm, acc_hbm,
          dst2_v, idx_a, idx_b, rows_0, rows_1, rows_2, rows_3, acc_sh,
          sem_g0, sem_g1, sem_g2, sem_g3,
          sem_s0, sem_s1, sem_s2, sem_s3, sem_ia, sem_ib):
    c = lax.axis_index("c")
    s = lax.axis_index("s")
    wid = s * NC + c
    pltpu.sync_copy(dst2_hbm.at[wid], dst2_v)

    def zrow(i, carry):
        r = i // 8
        k = i - r * 8
        rows_0[r, pl.ds(k * 16, 16)] = jnp.zeros((16,), jnp.float32)
        return carry

    lax.fori_loop(0, CH2 * 8, zrow, 0)
    row0 = s * RPS
    for t in range(RPS // CH2):
        pltpu.sync_copy(rows_0, acc_sh.at[pl.ds(row0 + t * CH2, CH2)])
    plsc.subcore_barrier()

    pltpu.async_copy(src2_hbm.at[wid, pl.ds(0, GRP)], idx_a, sem_ia)
    pltpu.async_copy(src2_hbm.at[wid, pl.ds(GRP, GRP)], idx_b, sem_ib)

    bufs = (rows_0, rows_1, rows_2, rows_3)
    gsems = (sem_g0, sem_g1, sem_g2, sem_g3)
    ssems = (sem_s0, sem_s1, sem_s2, sem_s3)

    def run_group(g, idx_v, isem, next_off):
        pltpu.make_async_copy(
            src2_hbm.at[wid, pl.ds(0, GRP)], idx_v, isem).wait()
        dg = [None, None, None, None]
        ds_ = [None, None, None, None]

        def gath(u, p):
            ch, hf = u // 2, u % 2
            return pltpu.async_copy(
                m_hbm.at[idx_v.at[ch, pl.ds(hf * CH2, CH2)]],
                bufs[p], gsems[p])

        for q in range(4):
            dg[q] = gath(q, q)
        for k in range(NSUB):
            p = k % 4
            dg[p].wait()
            ds_[p] = pltpu.async_copy(
                bufs[p],
                acc_sh.at[dst2_v.at[g * GRP + k // 2,
                                    pl.ds((k % 2) * CH2, CH2)]],
                ssems[p], add=True)
            if k + 4 < NSUB:
                ds_[p].wait()
                dg[p] = gath(k + 4, p)
        pltpu.async_copy(src2_hbm.at[wid, pl.ds(next_off, GRP)], idx_v, isem)
        for q in range(4):
            ds_[q].wait()

    def pair_body(t, carry):
        run_group(2 * t, idx_a, sem_ia, (2 * t + 2) * GRP)
        run_group(2 * t + 1, idx_b, sem_ib, (2 * t + 3) * GRP)
        return carry

    lax.fori_loop(0, NGRP // 2, pair_body, 0)
    pltpu.make_async_copy(
        src2_hbm.at[wid, pl.ds(0, GRP)], idx_a, sem_ia).wait()
    pltpu.make_async_copy(
        src2_hbm.at[wid, pl.ds(0, GRP)], idx_b, sem_ib).wait()
    plsc.subcore_barrier()

    for t in range(RPS // CH2):
        pltpu.sync_copy(acc_sh.at[pl.ds(row0 + t * CH2, CH2)], rows_0)
        pltpu.sync_copy(rows_0, acc_hbm.at[c, pl.ds(row0 + t * CH2, CH2)])


def _dinv(degp):
    ones = jnp.ones((NT, 1), jnp.float32)
    deg = lax.dot_general(degp, ones, (((0,), (0,)), ((), ()))) + 1.0
    return lax.rsqrt(deg)


def _tc_first_body(degp_ref, x_ref, w_ref, out_ref):
    dinv = _dinv(degp_ref[...])
    g = lax.dot_general(x_ref[...], w_ref[...], (((1,), (1,)), ((), ())))
    out_ref[...] = g * dinv


_tc_first = pl.pallas_call(
    _tc_first_body, out_shape=jax.ShapeDtypeStruct((NPAD, D), jnp.float32))


def _tc_mid_body(degp_ref, acc_ref, m_ref, b_ref, w_ref, out_ref):
    dinv = _dinv(degp_ref[...])
    acc = acc_ref[0] + acc_ref[1]
    pre = dinv * (acc + m_ref[...]) + b_ref[...]
    rows = lax.broadcasted_iota(jnp.int32, (NPAD, 1), 0)
    h = jnp.where(rows < NN, jnp.maximum(pre, 0.0), 0.0)
    g = lax.dot_general(h, w_ref[...], (((1,), (1,)), ((), ())))
    out_ref[...] = g * dinv


_tc_mid = pl.pallas_call(
    _tc_mid_body, out_shape=jax.ShapeDtypeStruct((NPAD, D), jnp.float32))


def _tc_final_body(degp_ref, acc_ref, m_ref, b_ref, batch_ref, xin_ref,
                   wn_ref, bn_ref, wx_ref, bx_ref, out_ref):
    dinv = _dinv(degp_ref[...])
    acc = acc_ref[0] + acc_ref[1]
    pre = dinv * (acc + m_ref[...]) + b_ref[...]
    rows = lax.broadcasted_iota(jnp.int32, (NPAD, 1), 0)
    h = jnp.where(rows < NN, jnp.maximum(pre, 0.0), 0.0)

    cols = lax.broadcasted_iota(jnp.int32, (NPAD, NG), 1)
    p = jnp.where(batch_ref[...] == cols, jnp.float32(1.0), jnp.float32(0.0))
    sums = lax.dot_general(p, h, (((0,), (0,)), ((), ())))
    ones = jnp.ones((NPAD, 1), jnp.float32)
    cnt = lax.dot_general(p, ones, (((0,), (0,)), ((), ())))
    pooled = sums / jnp.maximum(cnt, 1.0)

    z = lax.dot_general(pooled, wn_ref[...], (((1,), (1,)), ((), ())))
    z = jnp.maximum(z + bn_ref[...], 0.0)
    a0 = jnp.sum(z * wx_ref[0:1, :], axis=1, keepdims=True) + bx_ref[0:1, 0:1]
    nn1 = jnp.sum(z * wx_ref[1:2, :], axis=1, keepdims=True) + bx_ref[0:1, 1:2]
    out_ref[...] = xin_ref[...] * (1.0 + nn1) - a0


_tc_final = pl.pallas_call(
    _tc_final_body, out_shape=jax.ShapeDtypeStruct((NG, 1), jnp.float32))


def kernel(x_in, x, edge_index, batch, W1, b1, W2, b2, Wn, bn, Wx, bx):
    x_pad = jnp.pad(x, ((0, NPAD - NN), (0, 0)))
    batch_pad = jnp.pad(batch, (0, NPAD - NN),
                        constant_values=NG).reshape(NPAD, 1)
    b1r = b1.reshape(1, D)
    b2r = b2.reshape(1, D)
    bnr = bn.reshape(1, D)
    bxr = bx.reshape(1, 2)

    src2, dst2, degp = _prep(edge_index[0], edge_index[1])
    m1 = _tc_first(degp, x_pad, W1)
    acc1 = _spmm(m1, src2, dst2)
    m2 = _tc_mid(degp, acc1, m1, b1r, W2)
    acc2 = _spmm(m2, src2, dst2)
    return _tc_final(degp, acc2, m2, b2r, batch_pad, x_in, Wn, bnr, Wx, bxr)


# speedup vs baseline: 1.0859x; 1.0012x over previous
import functools

import jax
import jax.numpy as jnp
from jax import lax
from jax.experimental import pallas as pl
from jax.experimental.pallas import tpu as pltpu
from jax.experimental.pallas import tpu_sc as plsc

NN = 10000
NPAD = 10240
NE = 320000
D = 128
NG = 64
NC, NS = 2, 16
NT = NC * NS
EPT = NE // NT
CHUNK = 128
CPT = 80
NGROUP = CPT * CHUNK // 16
CPT_PAD = 96
RPS = NPAD // NS

_mesh = plsc.VectorSubcoreMesh(
    core_axis_name="c", subcore_axis_name="s", num_cores=NC, num_subcores=NS)


@functools.partial(
    pl.kernel,
    out_type=(
        jax.ShapeDtypeStruct((NT, CPT_PAD, CHUNK), jnp.int32),
        jax.ShapeDtypeStruct((NT, CPT, CHUNK), jnp.int32),
        jax.ShapeDtypeStruct((NT, NPAD), jnp.float32),
    ),
    mesh=_mesh,
    scratch_types=[
        pltpu.VMEM((EPT,), jnp.int32),
        pltpu.VMEM((EPT,), jnp.int32),
        pltpu.VMEM((CPT, CHUNK), jnp.int32),
        pltpu.VMEM((CPT, CHUNK), jnp.int32),
        pltpu.VMEM((NPAD,), jnp.float32),
    ],
    compiler_params=pltpu.CompilerParams(needs_layout_passes=False),
)
def _prep(esrc_hbm, edst_hbm, src2_hbm, dst2_hbm, degp_hbm,
          src_v, dst_v, src2_v, dst2_v, deg_v):
    c = lax.axis_index("c")
    s = lax.axis_index("s")
    wid = s * NC + c
    base = wid * EPT
    pltpu.sync_copy(esrc_hbm.at[pl.ds(base, EPT)], src_v)
    pltpu.sync_copy(edst_hbm.at[pl.ds(base, EPT)], dst_v)

    def zero_body(i, carry):
        deg_v[pl.ds(i * 16, 16)] = jnp.zeros((16,), jnp.float32)
        return carry

    lax.fori_loop(0, NPAD // 16, zero_body, 0)

    lanes = lax.iota(jnp.int32, 16)

    def grp_body(g, carry):
        jc = g // 8
        k = g - jc * 8
        eidx = g * 16 + lanes
        valid = eidx < EPT
        off = jnp.where(g * 16 < EPT, g * 16, 0)
        sv = src_v[pl.ds(off, 16)]
        dv = dst_v[pl.ds(off, 16)]
        keep = jnp.logical_and(valid, sv != dv)
        w = jnp.where(keep, jnp.float32(1.0), jnp.float32(0.0))
        d2 = jnp.where(valid, dv, jnp.int32(NN))
        s2 = jnp.where(keep, sv, jnp.int32(NN))
        plsc.addupdate_scatter(deg_v, [d2], w)
        src2_v[jc, pl.ds(k * 16, 16)] = s2
        dst2_v[jc, pl.ds(k * 16, 16)] = d2
        return carry

    lax.fori_loop(0, NGROUP, grp_body, 0)

    pltpu.sync_copy(src2_v, src2_hbm.at[wid, pl.ds(0, CPT)])
    pltpu.sync_copy(dst2_v, dst2_hbm.at[wid])
    pltpu.sync_copy(deg_v, degp_hbm.at[wid])


GRP = 8
NGRP = CPT // GRP
CH2 = 32
NSUB = GRP * CHUNK // CH2
NBUF = 8


@functools.partial(
    pl.kernel,
    out_type=jax.ShapeDtypeStruct((NC, NPAD, D), jnp.float32),
    mesh=_mesh,
    scratch_types=[
        pltpu.VMEM((CPT, CHUNK), jnp.int32),
        pltpu.VMEM((GRP, CHUNK), jnp.int32),
        pltpu.VMEM((GRP, CHUNK), jnp.int32),
        pltpu.VMEM((CH2, D), jnp.float32),
        pltpu.VMEM((CH2, D), jnp.float32),
        pltpu.VMEM((CH2, D), jnp.float32),
        pltpu.VMEM((CH2, D), jnp.float32),
        pltpu.VMEM((CH2, D), jnp.float32),
        pltpu.VMEM((CH2, D), jnp.float32),
        pltpu.VMEM((CH2, D), jnp.float32),
        pltpu.VMEM((CH2, D), jnp.float32),
        pltpu.VMEM_SHARED((NPAD, D), jnp.float32),
        pltpu.SemaphoreType.DMA,
        pltpu.SemaphoreType.DMA,
        pltpu.SemaphoreType.DMA,
        pltpu.SemaphoreType.DMA,
        pltpu.SemaphoreType.DMA,
        pltpu.SemaphoreType.DMA,
        pltpu.SemaphoreType.DMA,
        pltpu.SemaphoreType.DMA,
        pltpu.SemaphoreType.DMA,
        pltpu.SemaphoreType.DMA,
        pltpu.SemaphoreType.DMA,
        pltpu.SemaphoreType.DMA,
        pltpu.SemaphoreType.DMA,
        pltpu.SemaphoreType.DMA,
        pltpu.SemaphoreType.DMA,
        pltpu.SemaphoreType.DMA,
        pltpu.SemaphoreType.DMA,
        pltpu.SemaphoreType.DMA,
    ],
)
def _spmm(m_hbm, src2_hbm, dst2_hbm, acc_hbm,
          dst2_v, idx_a, idx_b,
          rows_0, rows_1, rows_2, rows_3, rows_4, rows_5, rows_6, rows_7,
          acc_sh,
          sem_g0, sem_g1, sem_g2, sem_g3, sem_g4, sem_g5, sem_g6, sem_g7,
          sem_s0, sem_s1, sem_s2, sem_s3, sem_s4, sem_s5, sem_s6, sem_s7,
          sem_ia, sem_ib):
    c = lax.axis_index("c")
    s = lax.axis_index("s")
    wid = s * NC + c
    pltpu.sync_copy(dst2_hbm.at[wid], dst2_v)

    def zrow(i, carry):
        r = i // 8
        k = i - r * 8
        rows_0[r, pl.ds(k * 16, 16)] = jnp.zeros((16,), jnp.float32)
        return carry

    lax.fori_loop(0, CH2 * 8, zrow, 0)
    row0 = s * RPS
    for t in range(RPS // CH2):
        pltpu.sync_copy(rows_0, acc_sh.at[pl.ds(row0 + t * CH2, CH2)])
    plsc.subcore_barrier()

    pltpu.async_copy(src2_hbm.at[wid, pl.ds(0, GRP)], idx_a, sem_ia)
    pltpu.async_copy(src2_hbm.at[wid, pl.ds(GRP, GRP)], idx_b, sem_ib)

    bufs = (rows_0, rows_1, rows_2, rows_3, rows_4, rows_5, rows_6, rows_7)
    gsems = (sem_g0, sem_g1, sem_g2, sem_g3, sem_g4, sem_g5, sem_g6, sem_g7)
    ssems = (sem_s0, sem_s1, sem_s2, sem_s3, sem_s4, sem_s5, sem_s6, sem_s7)
    SUBQ = CHUNK // CH2

    def run_group(g, idx_v, isem, next_off):
        pltpu.make_async_copy(
            src2_hbm.at[wid, pl.ds(0, GRP)], idx_v, isem).wait()
        dg = [None] * NBUF
        ds_ = [None] * NBUF

        def gath(u, p):
            ch, q = u // SUBQ, u % SUBQ
            return pltpu.async_copy(
                m_hbm.at[idx_v.at[ch, pl.ds(q * CH2, CH2)]],
                bufs[p], gsems[p])

        for q in range(NBUF):
            dg[q] = gath(q, q)
        for k in range(NSUB):
            p = k % NBUF
            dg[p].wait()
            ds_[p] = pltpu.async_copy(
                bufs[p],
                acc_sh.at[dst2_v.at[g * GRP + k // SUBQ,
                                    pl.ds((k % SUBQ) * CH2, CH2)]],
                ssems[p], add=True)
            if k + NBUF < NSUB:
                ds_[p].wait()
                dg[p] = gath(k + NBUF, p)
        pltpu.async_copy(src2_hbm.at[wid, pl.ds(next_off, GRP)], idx_v, isem)
        for q in range(NBUF):
            ds_[q].wait()

    def pair_body(t, carry):
        run_group(2 * t, idx_a, sem_ia, (2 * t + 2) * GRP)
        run_group(2 * t + 1, idx_b, sem_ib, (2 * t + 3) * GRP)
        return carry

    lax.fori_loop(0, NGRP // 2, pair_body, 0)
    pltpu.make_async_copy(
        src2_hbm.at[wid, pl.ds(0, GRP)], idx_a, sem_ia).wait()
    pltpu.make_async_copy(
        src2_hbm.at[wid, pl.ds(0, GRP)], idx_b, sem_ib).wait()
    plsc.subcore_barrier()

    for t in range(RPS // CH2):
        pltpu.sync_copy(acc_sh.at[pl.ds(row0 + t * CH2, CH2)], rows_0)
        pltpu.sync_copy(rows_0, acc_hbm.at[c, pl.ds(row0 + t * CH2, CH2)])


def _dinv(degp):
    ones = jnp.ones((NT, 1), jnp.float32)
    deg = lax.dot_general(degp, ones, (((0,), (0,)), ((), ()))) + 1.0
    return lax.rsqrt(deg)


def _tc_first_body(degp_ref, x_ref, w_ref, out_ref):
    dinv = _dinv(degp_ref[...])
    g = lax.dot_general(x_ref[...], w_ref[...], (((1,), (1,)), ((), ())))
    out_ref[...] = g * dinv


_tc_first = pl.pallas_call(
    _tc_first_body, out_shape=jax.ShapeDtypeStruct((NPAD, D), jnp.float32))


def _tc_mid_body(degp_ref, acc_ref, m_ref, b_ref, w_ref, out_ref):
    dinv = _dinv(degp_ref[...])
    acc = acc_ref[0] + acc_ref[1]
    pre = dinv * (acc + m_ref[...]) + b_ref[...]
    rows = lax.broadcasted_iota(jnp.int32, (NPAD, 1), 0)
    h = jnp.where(rows < NN, jnp.maximum(pre, 0.0), 0.0)
    g = lax.dot_general(h, w_ref[...], (((1,), (1,)), ((), ())))
    out_ref[...] = g * dinv


_tc_mid = pl.pallas_call(
    _tc_mid_body, out_shape=jax.ShapeDtypeStruct((NPAD, D), jnp.float32))


def _tc_final_body(degp_ref, acc_ref, m_ref, b_ref, batch_ref, xin_ref,
                   wn_ref, bn_ref, wx_ref, bx_ref, out_ref):
    dinv = _dinv(degp_ref[...])
    acc = acc_ref[0] + acc_ref[1]
    pre = dinv * (acc + m_ref[...]) + b_ref[...]
    rows = lax.broadcasted_iota(jnp.int32, (NPAD, 1), 0)
    h = jnp.where(rows < NN, jnp.maximum(pre, 0.0), 0.0)

    cols = lax.broadcasted_iota(jnp.int32, (NPAD, NG), 1)
    p = jnp.where(batch_ref[...] == cols, jnp.float32(1.0), jnp.float32(0.0))
    sums = lax.dot_general(p, h, (((0,), (0,)), ((), ())))
    ones = jnp.ones((NPAD, 1), jnp.float32)
    cnt = lax.dot_general(p, ones, (((0,), (0,)), ((), ())))
    pooled = sums / jnp.maximum(cnt, 1.0)

    z = lax.dot_general(pooled, wn_ref[...], (((1,), (1,)), ((), ())))
    z = jnp.maximum(z + bn_ref[...], 0.0)
    a0 = jnp.sum(z * wx_ref[0:1, :], axis=1, keepdims=True) + bx_ref[0:1, 0:1]
    nn1 = jnp.sum(z * wx_ref[1:2, :], axis=1, keepdims=True) + bx_ref[0:1, 1:2]
    out_ref[...] = xin_ref[...] * (1.0 + nn1) - a0


_tc_final = pl.pallas_call(
    _tc_final_body, out_shape=jax.ShapeDtypeStruct((NG, 1), jnp.float32))


def kernel(x_in, x, edge_index, batch, W1, b1, W2, b2, Wn, bn, Wx, bx):
    x_pad = jnp.pad(x, ((0, NPAD - NN), (0, 0)))
    batch_pad = jnp.pad(batch, (0, NPAD - NN),
                        constant_values=NG).reshape(NPAD, 1)
    b1r = b1.reshape(1, D)
    b2r = b2.reshape(1, D)
    bnr = bn.reshape(1, D)
    bxr = bx.reshape(1, 2)

    src2, dst2, degp = _prep(edge_index[0], edge_index[1])
    m1 = _tc_first(degp, x_pad, W1)
    acc1 = _spmm(m1, src2, dst2)
    m2 = _tc_mid(degp, acc1, m1, b1r, W2)
    acc2 = _spmm(m2, src2, dst2)
    return _tc_final(degp, acc2, m2, b2r, batch_pad, x_in, Wn, bnr, Wx, bxr)
